# Initial kernel scaffold; baseline (speedup 1.0000x reference)
import jax
import jax.numpy as jnp
from jax.experimental import pallas as pl


def kernel(buf, dWt, t):
    raise NotImplementedError("write your pallas kernel here")


# speedup vs baseline: 1.0617x; 1.0617x over previous
import jax
import jax.numpy as jnp
from jax.experimental import pallas as pl
from jax.experimental.pallas import tpu as pltpu

_NH = 1024
_DT = 0.1
_DELAY = 512
_K = 0.1

_ROWS = 2048
_COLS = 32768
_C = 1024
_GRID = _COLS // _C


def _body(ts_ref, buf_ref, w_ref, outb_ref, outnx_ref):
    ts = ts_ref[0]
    outb_ref[...] = buf_ref[...]
    x = buf_ref[_NH + ts, :]
    a = buf_ref[_NH + ts - _DELAY, :]
    b = buf_ref[_NH + ts + 1 - _DELAY, :]
    w = w_ref[...]
    d1 = -x + _K * jnp.tanh(a)
    xi = x + _DT * d1 + w
    d2 = -xi + _K * jnp.tanh(b)
    nx = x + _DT * 0.5 * (d1 + d2) + w
    outnx_ref[...] = nx
    outb_ref[_NH + ts + 1, :] = nx


def kernel(buf, dWt, t):
    ts = t[0, 0:1].astype(jnp.int32)
    grid_spec = pltpu.PrefetchScalarGridSpec(
        num_scalar_prefetch=1,
        grid=(_GRID,),
        in_specs=[
            pl.BlockSpec((_ROWS, _C), lambda i, ts: (0, i)),
            pl.BlockSpec((_C,), lambda i, ts: (i,)),
        ],
        out_specs=[
            pl.BlockSpec((_ROWS, _C), lambda i, ts: (0, i)),
            pl.BlockSpec((_C,), lambda i, ts: (i,)),
        ],
    )
    buf2, nx = pl.pallas_call(
        _body,
        grid_spec=grid_spec,
        out_shape=[
            jax.ShapeDtypeStruct((_ROWS, _COLS), jnp.float32),
            jax.ShapeDtypeStruct((_COLS,), jnp.float32),
        ],
    )(ts, buf, dWt)
    return (buf2, nx)
